# baseline (speedup 1.0000x reference)
import functools

import jax
import jax.numpy as jnp
from jax import lax
from jax.experimental import pallas as pl
from jax.experimental.pallas import tpu as pltpu
from jax.experimental.pallas import tpu_sc as plsc

N = 10000
E = 320000
D = 128
L = 3

NC = 2
NS = 16
NW = NC * NS
EPT = E // NW
CH = 80
NCHUNK = EPT // CH
NP = 10240
RPS = NP // NS
ZROWS = 128
DEG_W = 16

DH = D // NC
EPC = E // NS
NCHUNK2 = EPC // CH
DEPTH = 5
NSLOT = DEPTH

def _zero_fill(ref, rows, width):
    @pl.loop(0, rows)
    def _(r):
        @pl.loop(0, width, step=16)
        def _(c0):
            ref[r, pl.ds(c0, 16)] = jnp.zeros((16,), jnp.float32)


@functools.cache
def _make_sc_segment_sum():
    mesh = plsc.VectorSubcoreMesh(core_axis_name="c", subcore_axis_name="s")

    @functools.partial(
        pl.kernel,
        out_type=jax.ShapeDtypeStruct((NC, NP, DH), jnp.float32),
        mesh=mesh,
        compiler_params=pltpu.CompilerParams(use_tc_tiling_on_sc=False),
        scratch_types=[
            pltpu.VMEM((NCHUNK2, CH), jnp.int32),
            pltpu.VMEM((NCHUNK2, CH), jnp.int32),
            [pltpu.VMEM((CH, DH), jnp.float32)] * NSLOT,
            pltpu.VMEM((ZROWS, DH), jnp.float32),
            pltpu.VMEM_SHARED((NP, DH), jnp.float32),
            pltpu.SemaphoreType.DMA,
            pltpu.SemaphoreType.DMA,
            [pltpu.SemaphoreType.DMA] * NSLOT,
        ],
    )
    def sc_segment_sum(h_hbm, src_hbm, dst_hbm, out_hbm,
                       src_v, dst_v, gbufs, zbuf, acc, isem0, isem1,
                       gsems):
        c = lax.axis_index("c")
        s = lax.axis_index("s")
        w = c * NS + s
        d_src = pltpu.async_copy(src_hbm.at[w], src_v, isem0)
        d_dst = pltpu.async_copy(dst_hbm.at[s], dst_v, isem1)
        _zero_fill(zbuf, ZROWS, DH)

        @pl.loop(0, RPS, step=ZROWS)
        def _(r0):
            pltpu.sync_copy(zbuf, acc.at[pl.ds(s * RPS + r0, ZROWS)])

        d_src.wait()
        d_dst.wait()
        plsc.subcore_barrier()

        for b in range(DEPTH):
            pltpu.async_copy(h_hbm.at[src_v.at[b]], gbufs[b], gsems[b])

        @pl.loop(0, NCHUNK2, step=NSLOT)
        def _(g):
            for b in range(NSLOT):
                j = g + b
                pltpu.make_async_copy(h_hbm.at[src_v.at[j]],
                                      gbufs[b], gsems[b]).wait()
                pltpu.sync_copy(gbufs[b], acc.at[dst_v.at[j]], add=True)

                @pl.when(j + DEPTH < NCHUNK2)
                def _():
                    pltpu.async_copy(h_hbm.at[src_v.at[j + DEPTH]],
                                     gbufs[b], gsems[b])

        plsc.subcore_barrier()
        pltpu.sync_copy(acc.at[pl.ds(s * RPS, RPS)],
                        out_hbm.at[c].at[pl.ds(s * RPS, RPS)])

    return sc_segment_sum


@functools.cache
def _make_sc_degree():
    mesh = plsc.VectorSubcoreMesh(core_axis_name="c", subcore_axis_name="s")

    @functools.partial(
        pl.kernel,
        out_type=jax.ShapeDtypeStruct((NW, NP), jnp.float32),
        mesh=mesh,
        compiler_params=pltpu.CompilerParams(use_tc_tiling_on_sc=False,
                                             needs_layout_passes=False),
        scratch_types=[
            pltpu.VMEM((EPT,), jnp.int32),
            pltpu.VMEM((NP,), jnp.float32),
            pltpu.SemaphoreType.DMA,
        ],
    )
    def sc_degree(dst_hbm, out_hbm, dst_v, acc_v, isem):
        c = lax.axis_index("c")
        s = lax.axis_index("s")
        w = c * NS + s
        d_dst = pltpu.async_copy(dst_hbm.at[w], dst_v, isem)

        @pl.loop(0, NP, step=16)
        def _(i):
            acc_v[pl.ds(i, 16)] = jnp.zeros((16,), jnp.float32)

        d_dst.wait()
        ones = jnp.ones((16,), jnp.float32)

        @pl.loop(0, EPT, step=16)
        def _(i):
            plsc.addupdate_scatter(acc_v, [dst_v[pl.ds(i, 16)]], ones)

        pltpu.sync_copy(acc_v, out_hbm.at[w])

    return sc_degree


_BN = 1024


def _tc_encoder(x, W, b):
    def body(x_ref, w_ref, b_ref, o_ref, o2_ref):
        z = jnp.dot(x_ref[...], w_ref[...], preferred_element_type=jnp.float32)
        z = jnp.maximum(z + b_ref[...], 0.0)
        o_ref[...] = z
        o2_ref[0] = z[:, :DH]
        o2_ref[1] = z[:, DH:]

    return pl.pallas_call(
        body,
        grid=(pl.cdiv(N, _BN),),
        in_specs=[
            pl.BlockSpec((_BN, D), lambda i: (i, 0)),
            pl.BlockSpec((D, D), lambda i: (0, 0)),
            pl.BlockSpec((1, D), lambda i: (0, 0)),
        ],
        out_specs=[pl.BlockSpec((_BN, D), lambda i: (i, 0)),
                   pl.BlockSpec((NC, _BN, DH), lambda i: (0, i, 0))],
        out_shape=[jax.ShapeDtypeStruct((N, D), jnp.float32),
                   jax.ShapeDtypeStruct((NC, N, DH), jnp.float32)],
    )(x, W, b.reshape(1, D))


def _tc_layer(h, part, degp, Ws, Wn, b, Wh=None, bh=None):
    fuse_head = Wh is not None

    def body(h_ref, p_ref, g_ref, ws_ref, wn_ref, b_ref, *rest):
        h_blk = h_ref[...]
        psum = jnp.concatenate([p_ref[0], p_ref[1]], axis=1)
        deg = jnp.sum(g_ref[...], axis=0)[:, None]
        agg = psum * (1.0 / jnp.maximum(deg, 1.0))
        z = (jnp.dot(h_blk, ws_ref[...], preferred_element_type=jnp.float32)
             + jnp.dot(agg, wn_ref[...], preferred_element_type=jnp.float32)
             + b_ref[...])
        hn = jnp.maximum(z, 0.0) + h_blk
        if fuse_head:
            wh_ref, bh_ref, o_ref = rest
            o_ref[...] = (jnp.dot(hn, wh_ref[...],
                                  preferred_element_type=jnp.float32)
                          + bh_ref[...])
        else:
            o_ref, o2_ref = rest
            o_ref[...] = hn
            o2_ref[0] = hn[:, :DH]
            o2_ref[1] = hn[:, DH:]

    in_specs = [
        pl.BlockSpec((_BN, D), lambda i: (i, 0)),
        pl.BlockSpec((NC, _BN, DH), lambda i: (0, i, 0)),
        pl.BlockSpec((NW, _BN), lambda i: (0, i)),
        pl.BlockSpec((D, D), lambda i: (0, 0)),
        pl.BlockSpec((D, D), lambda i: (0, 0)),
        pl.BlockSpec((1, D), lambda i: (0, 0)),
    ]
    args = [h, part, degp, Ws, Wn, b.reshape(1, D)]
    if fuse_head:
        in_specs += [pl.BlockSpec((D, D), lambda i: (0, 0)),
                     pl.BlockSpec((1, D), lambda i: (0, 0))]
        args += [Wh, bh.reshape(1, D)]
        out_specs = pl.BlockSpec((_BN, D), lambda i: (i, 0))
        out_shape = jax.ShapeDtypeStruct((N, D), jnp.float32)
    else:
        out_specs = [pl.BlockSpec((_BN, D), lambda i: (i, 0)),
                     pl.BlockSpec((NC, _BN, DH), lambda i: (0, i, 0))]
        out_shape = [jax.ShapeDtypeStruct((N, D), jnp.float32),
                     jax.ShapeDtypeStruct((NC, N, DH), jnp.float32)]

    return pl.pallas_call(
        body,
        grid=(pl.cdiv(N, _BN),),
        in_specs=in_specs,
        out_specs=out_specs,
        out_shape=out_shape,
    )(*args)


def kernel(x, edge_index, W_enc, b_enc, W_self, W_neigh, b_conv, W_head, b_head):
    ei = edge_index.astype(jnp.int32)
    dst_deg = ei[1].reshape(NW, EPT)
    srcr = ei[0].reshape(NS, NCHUNK2, CH)
    src2 = jnp.stack([srcr, srcr + N]).reshape(NW, NCHUNK2, CH)
    dst_seg = ei[1].reshape(NS, NCHUNK2, CH)

    degp = _make_sc_degree()(dst_deg)
    x, degp = lax.optimization_barrier((x, degp))
    h, h2 = _tc_encoder(x, W_enc, b_enc)
    for l in range(L):
        part = _make_sc_segment_sum()(h2.reshape(2 * N, DH), src2, dst_seg)
        if l == L - 1:
            h = _tc_layer(h, part, degp, W_self[l], W_neigh[l], b_conv[l],
                          W_head, b_head)
        else:
            h, h2 = _tc_layer(h, part, degp, W_self[l], W_neigh[l], b_conv[l])
    return h


# speedup vs baseline: 1.1895x; 1.1895x over previous
import functools

import jax
import jax.numpy as jnp
from jax import lax
from jax.experimental import pallas as pl
from jax.experimental.pallas import tpu as pltpu
from jax.experimental.pallas import tpu_sc as plsc

N = 10000
E = 320000
D = 128
L = 3

NC = 2
NS = 16
NW = NC * NS
EPT = E // NW
CH = 80
NCHUNK = EPT // CH
NP = 10240
RPS = NP // NS
ZROWS = 128
DEG_W = 16

DH = D // NC
EPC = E // NS
NCHUNK2 = EPC // CH
DEPTH = 5
NSLOT = DEPTH

def _zero_fill(ref, rows, width):
    @pl.loop(0, rows)
    def _(r):
        @pl.loop(0, width, step=16)
        def _(c0):
            ref[r, pl.ds(c0, 16)] = jnp.zeros((16,), jnp.float32)


@functools.cache
def _make_sc_segment_sum():
    mesh = plsc.VectorSubcoreMesh(core_axis_name="c", subcore_axis_name="s")

    @functools.partial(
        pl.kernel,
        out_type=jax.ShapeDtypeStruct((NP, D), jnp.float32),
        mesh=mesh,
        compiler_params=pltpu.CompilerParams(use_tc_tiling_on_sc=False),
        scratch_types=[
            pltpu.VMEM((NCHUNK2, CH), jnp.int32),
            pltpu.VMEM((NCHUNK2, CH), jnp.int32),
            [pltpu.VMEM((CH, DH), jnp.float32)] * NSLOT,
            pltpu.VMEM((ZROWS, DH), jnp.float32),
            pltpu.VMEM_SHARED((NP, DH), jnp.float32),
            pltpu.SemaphoreType.DMA,
            pltpu.SemaphoreType.DMA,
            [pltpu.SemaphoreType.DMA] * NSLOT,
        ],
    )
    def sc_segment_sum(h_hbm, src_hbm, dst_hbm, out_hbm,
                       src_v, dst_v, gbufs, zbuf, acc, isem0, isem1,
                       gsems):
        c = lax.axis_index("c")
        s = lax.axis_index("s")
        w = c * NS + s
        d_src = pltpu.async_copy(src_hbm.at[w], src_v, isem0)
        d_dst = pltpu.async_copy(dst_hbm.at[s], dst_v, isem1)
        _zero_fill(zbuf, ZROWS, DH)

        @pl.loop(0, RPS, step=ZROWS)
        def _(r0):
            pltpu.sync_copy(zbuf, acc.at[pl.ds(s * RPS + r0, ZROWS)])

        d_src.wait()
        d_dst.wait()
        plsc.subcore_barrier()

        for b in range(DEPTH):
            pltpu.async_copy(h_hbm.at[src_v.at[b]], gbufs[b], gsems[b])

        @pl.loop(0, NCHUNK2, step=NSLOT)
        def _(g):
            for b in range(NSLOT):
                j = g + b
                pltpu.make_async_copy(h_hbm.at[src_v.at[j]],
                                      gbufs[b], gsems[b]).wait()
                pltpu.sync_copy(gbufs[b], acc.at[dst_v.at[j]], add=True)

                @pl.when(j + DEPTH < NCHUNK2)
                def _():
                    pltpu.async_copy(h_hbm.at[src_v.at[j + DEPTH]],
                                     gbufs[b], gsems[b])

        plsc.subcore_barrier()
        pltpu.sync_copy(acc.at[pl.ds(s * RPS, RPS)],
                        out_hbm.at[pl.ds(s * RPS, RPS), pl.ds(c * DH, DH)])

    return sc_segment_sum


@functools.cache
def _make_sc_degree():
    mesh = plsc.VectorSubcoreMesh(core_axis_name="c", subcore_axis_name="s")

    @functools.partial(
        pl.kernel,
        out_type=jax.ShapeDtypeStruct((NW, NP), jnp.float32),
        mesh=mesh,
        compiler_params=pltpu.CompilerParams(use_tc_tiling_on_sc=False,
                                             needs_layout_passes=False),
        scratch_types=[
            pltpu.VMEM((EPT,), jnp.int32),
            pltpu.VMEM((NP,), jnp.float32),
            pltpu.SemaphoreType.DMA,
        ],
    )
    def sc_degree(dst_hbm, out_hbm, dst_v, acc_v, isem):
        c = lax.axis_index("c")
        s = lax.axis_index("s")
        w = c * NS + s
        d_dst = pltpu.async_copy(dst_hbm.at[w], dst_v, isem)

        @pl.loop(0, NP, step=16)
        def _(i):
            acc_v[pl.ds(i, 16)] = jnp.zeros((16,), jnp.float32)

        d_dst.wait()
        ones = jnp.ones((16,), jnp.float32)

        @pl.loop(0, EPT, step=16)
        def _(i):
            plsc.addupdate_scatter(acc_v, [dst_v[pl.ds(i, 16)]], ones)

        pltpu.sync_copy(acc_v, out_hbm.at[w])

    return sc_degree


_BN = 1024


def _tc_encoder(x, W, b):
    def body(x_ref, w_ref, b_ref, o_ref):
        z = jnp.dot(x_ref[...], w_ref[...], preferred_element_type=jnp.float32)
        o_ref[...] = jnp.maximum(z + b_ref[...], 0.0)

    return pl.pallas_call(
        body,
        grid=(pl.cdiv(N, _BN),),
        in_specs=[
            pl.BlockSpec((_BN, D), lambda i: (i, 0)),
            pl.BlockSpec((D, D), lambda i: (0, 0)),
            pl.BlockSpec((1, D), lambda i: (0, 0)),
        ],
        out_specs=pl.BlockSpec((_BN, D), lambda i: (i, 0)),
        out_shape=jax.ShapeDtypeStruct((N, D), jnp.float32),
    )(x, W, b.reshape(1, D))


def _tc_layer(h, part, degp, Ws, Wn, b, Wh=None, bh=None):
    fuse_head = Wh is not None

    def body(h_ref, p_ref, g_ref, ws_ref, wn_ref, b_ref, *rest):
        h_blk = h_ref[...]
        psum = p_ref[...]
        deg = jnp.sum(g_ref[...], axis=0)[:, None]
        agg = psum * (1.0 / jnp.maximum(deg, 1.0))
        z = (jnp.dot(h_blk, ws_ref[...], preferred_element_type=jnp.float32)
             + jnp.dot(agg, wn_ref[...], preferred_element_type=jnp.float32)
             + b_ref[...])
        hn = jnp.maximum(z, 0.0) + h_blk
        if fuse_head:
            wh_ref, bh_ref, o_ref = rest
            o_ref[...] = (jnp.dot(hn, wh_ref[...],
                                  preferred_element_type=jnp.float32)
                          + bh_ref[...])
        else:
            (o_ref,) = rest
            o_ref[...] = hn

    in_specs = [
        pl.BlockSpec((_BN, D), lambda i: (i, 0)),
        pl.BlockSpec((_BN, D), lambda i: (i, 0)),
        pl.BlockSpec((NW, _BN), lambda i: (0, i)),
        pl.BlockSpec((D, D), lambda i: (0, 0)),
        pl.BlockSpec((D, D), lambda i: (0, 0)),
        pl.BlockSpec((1, D), lambda i: (0, 0)),
    ]
    args = [h, part, degp, Ws, Wn, b.reshape(1, D)]
    if fuse_head:
        in_specs += [pl.BlockSpec((D, D), lambda i: (0, 0)),
                     pl.BlockSpec((1, D), lambda i: (0, 0))]
        args += [Wh, bh.reshape(1, D)]
    out_specs = pl.BlockSpec((_BN, D), lambda i: (i, 0))
    out_shape = jax.ShapeDtypeStruct((N, D), jnp.float32)

    return pl.pallas_call(
        body,
        grid=(pl.cdiv(N, _BN),),
        in_specs=in_specs,
        out_specs=out_specs,
        out_shape=out_shape,
    )(*args)


def kernel(x, edge_index, W_enc, b_enc, W_self, W_neigh, b_conv, W_head, b_head):
    ei = edge_index.astype(jnp.int32)
    dst_deg = ei[1].reshape(NW, EPT)
    src_flat = ei[0]
    src2 = jnp.stack([2 * src_flat, 2 * src_flat + 1]).reshape(NW, NCHUNK2, CH)
    dst_seg = ei[1].reshape(NS, NCHUNK2, CH)

    degp = _make_sc_degree()(dst_deg)
    x, degp = lax.optimization_barrier((x, degp))
    h = _tc_encoder(x, W_enc, b_enc)
    for l in range(L):
        part = _make_sc_segment_sum()(h.reshape(2 * N, DH), src2, dst_seg)
        if l == L - 1:
            h = _tc_layer(h, part, degp, W_self[l], W_neigh[l], b_conv[l],
                          W_head, b_head)
        else:
            h = _tc_layer(h, part, degp, W_self[l], W_neigh[l], b_conv[l])
    return h
